# Initial kernel scaffold; baseline (speedup 1.0000x reference)
import jax
import jax.numpy as jnp
from jax.experimental import pallas as pl


def kernel(tokens, embedding_weight):
    raise NotImplementedError("write your pallas kernel here")


# speedup vs baseline: 1.8301x; 1.8301x over previous
import functools

import jax
import jax.numpy as jnp
from jax import lax
from jax.experimental import pallas as pl
from jax.experimental.pallas import tpu as pltpu
from jax.experimental.pallas import tpu_sc as plsc

VOCAB_SIZE = 1000000
EMBED_DIM = 64
BATCH = 16384
HIST_LEN = 50

_INFO = plsc.get_sparse_core_info()
_NC, _NS = _INFO.num_cores, _INFO.num_subcores
_NW = _NC * _NS
_B = BATCH * HIST_LEN
_BPW = _B // _NW
_CH = 512
_NCHUNK = _BPW // _CH


def _make_sc_gather():
  mesh = plsc.VectorSubcoreMesh(core_axis_name="c", subcore_axis_name="s")

  @functools.partial(
      pl.kernel,
      mesh=mesh,
      compiler_params=pltpu.CompilerParams(use_tc_tiling_on_sc=False),
      out_type=jax.ShapeDtypeStruct((_B, EMBED_DIM), jnp.float32),
      scratch_types=[
          pltpu.VMEM((_BPW,), jnp.int32),
          pltpu.VMEM((_CH, EMBED_DIM), jnp.float32),
          pltpu.SemaphoreType.DMA,
      ],
  )
  def k(table_hbm, idx_hbm, out_hbm, idx_v, rows_v, sem):
    wid = lax.axis_index("s") * _NC + lax.axis_index("c")
    base = wid * _BPW
    pltpu.sync_copy(idx_hbm.at[pl.ds(base, _BPW)], idx_v)

    def body(g, _):
      off = g * _CH
      pltpu.async_copy(table_hbm.at[idx_v.at[pl.ds(off, _CH)]], rows_v,
                       sem).wait()
      pltpu.sync_copy(rows_v, out_hbm.at[pl.ds(base + off, _CH)])
      return 0

    lax.fori_loop(0, _NCHUNK, body, 0)

  return k


_sc_gather = _make_sc_gather()


def kernel(tokens, embedding_weight):
  idx = tokens.reshape(_B).astype(jnp.int32)
  out = _sc_gather(embedding_weight, idx)
  return out.reshape(BATCH, HIST_LEN, EMBED_DIM)


# speedup vs baseline: 1.8751x; 1.0246x over previous
import functools

import jax
import jax.numpy as jnp
from jax import lax
from jax.experimental import pallas as pl
from jax.experimental.pallas import tpu as pltpu
from jax.experimental.pallas import tpu_sc as plsc

VOCAB_SIZE = 1000000
EMBED_DIM = 64
BATCH = 16384
HIST_LEN = 50

_INFO = plsc.get_sparse_core_info()
_NC, _NS = _INFO.num_cores, _INFO.num_subcores
_NW = _NC * _NS
_B = BATCH * HIST_LEN
_BPW = _B // _NW
_CH = 128
_NCHUNK = _BPW // _CH
_RING = 8
_LOOK = 4
_NSUP = _NCHUNK // _RING


def _make_sc_gather():
  mesh = plsc.VectorSubcoreMesh(core_axis_name="c", subcore_axis_name="s")

  @functools.partial(
      pl.kernel,
      mesh=mesh,
      compiler_params=pltpu.CompilerParams(use_tc_tiling_on_sc=False),
      out_type=jax.ShapeDtypeStruct((_B, EMBED_DIM), jnp.float32),
      scratch_types=[
          pltpu.VMEM((_BPW,), jnp.int32),
          pltpu.VMEM((_RING, _CH, EMBED_DIM), jnp.float32),
          [pltpu.SemaphoreType.DMA] * _RING,
          [pltpu.SemaphoreType.DMA] * _RING,
      ],
  )
  def k(table_hbm, idx_hbm, out_hbm, idx_v, rows_v, gsem, ssem):
    wid = lax.axis_index("s") * _NC + lax.axis_index("c")
    base = wid * _BPW
    pltpu.sync_copy(idx_hbm.at[pl.ds(base, _BPW)], idx_v)

    def g_copy(c, b):
      return pltpu.make_async_copy(
          table_hbm.at[idx_v.at[pl.ds(c * _CH, _CH)]], rows_v.at[b], gsem[b])

    def s_copy(c, b):
      return pltpu.make_async_copy(
          rows_v.at[b], out_hbm.at[pl.ds(base + c * _CH, _CH)], ssem[b])

    def step(c, b, launch):
      g_copy(c, b).wait()
      s_copy(c, b).start()
      if launch:
        nb = (b + _LOOK) % _RING
        if launch == 2:
          s_copy(0, nb).wait()
        g_copy(c + _LOOK, nb).start()

    for b in range(_LOOK):
      g_copy(b, b).start()

    for b in range(_RING):
      step(b, b, launch=1 if b + _LOOK < _RING else 2)

    def body(s, _):
      for b in range(_RING):
        step(s * _RING + b, b, launch=2)
      return 0

    lax.fori_loop(1, _NSUP - 1, body, 0)

    c0 = (_NSUP - 1) * _RING
    for b in range(_RING):
      step(c0 + b, b, launch=2 if c0 + b + _LOOK < _NCHUNK else 0)

    for b in range(_RING):
      s_copy(0, b).wait()

  return k


_sc_gather = _make_sc_gather()


def kernel(tokens, embedding_weight):
  idx = tokens.reshape(_B).astype(jnp.int32)
  out = _sc_gather(embedding_weight, idx)
  return out.reshape(BATCH, HIST_LEN, EMBED_DIM)
